# Initial kernel scaffold; baseline (speedup 1.0000x reference)
import jax
import jax.numpy as jnp
from jax.experimental import pallas as pl


def kernel(z_e, codebook):
    raise NotImplementedError("write your pallas kernel here")


# speedup vs baseline: 1.1417x; 1.1417x over previous
import functools

import jax
import jax.numpy as jnp
from jax import lax
from jax.experimental import pallas as pl
from jax.experimental.pallas import tpu as pltpu
from jax.experimental.pallas import tpu_sc as plsc

_BK = 2736
_KCB = 8192


def _bf16_rtne(x):
    u = lax.bitcast_convert_type(x, jnp.int32)
    r = (u + jnp.int32(0x7FFF) + ((u >> 16) & jnp.int32(1))) & jnp.int32(~0xFFFF)
    return lax.bitcast_convert_type(r, jnp.float32)


def _dist_argmin_kernel(
    zb_ref, cb_ref, fn_ref, cn_ref, idx_ref, loss_ref, minv, mina, accv
):
    b = pl.program_id(0)
    kt = pl.program_id(1)
    nk = pl.num_programs(1)
    nb = pl.num_programs(0)
    bk = cb_ref.shape[0]
    kcb = _KCB

    cross = jnp.dot(
        cb_ref[...], zb_ref[0], preferred_element_type=jnp.float32
    )
    dist = (fn_ref[0] - 2.0 * cross) + cn_ref[...]
    kglob = lax.broadcasted_iota(jnp.int32, dist.shape, 0) + kt * bk
    dist = jnp.where(kglob < kcb, dist, jnp.inf)
    tmin = jnp.min(dist, axis=0, keepdims=True)
    targ = jnp.min(
        jnp.where(dist == tmin, kglob, jnp.int32(2**30)), axis=0, keepdims=True
    )

    @pl.when(jnp.logical_and(b == 0, kt == 0))
    def _():
        accv[...] = jnp.zeros_like(accv)

    @pl.when(kt == 0)
    def _():
        minv[...] = _bf16_rtne(tmin)
        mina[...] = targ

    @pl.when(kt != 0)
    def _():
        upd = tmin < minv[...]
        merged = jnp.where(upd, tmin, minv[...])
        mina[...] = jnp.where(upd, targ, mina[...])
        minv[...] = _bf16_rtne(merged)

        @pl.when(kt == nk - 1)
        def _():
            idx_ref[0] = mina[...]
            accv[...] += merged

    @pl.when(jnp.logical_and(b == nb - 1, kt == nk - 1))
    def _():
        loss_ref[...] = accv[...]


def _nearest_indices(zb, cbb, fnorm, cnorm):
    B, D, HW = zb.shape
    K = cbb.shape[0]
    grid = (B, (K + _BK - 1) // _BK)
    return pl.pallas_call(
        _dist_argmin_kernel,
        grid=grid,
        in_specs=[
            pl.BlockSpec((1, D, HW), lambda b, kt: (b, 0, 0)),
            pl.BlockSpec((_BK, D), lambda b, kt: (kt, 0)),
            pl.BlockSpec((1, 1, HW), lambda b, kt: (b, 0, 0)),
            pl.BlockSpec((_BK, 1), lambda b, kt: (kt, 0)),
        ],
        out_specs=[
            pl.BlockSpec((1, 1, HW), lambda b, kt: (b, 0, 0)),
            pl.BlockSpec((1, HW), lambda b, kt: (0, 0)),
        ],
        out_shape=[
            jax.ShapeDtypeStruct((B, 1, HW), jnp.int32),
            jax.ShapeDtypeStruct((1, HW), jnp.float32),
        ],
        scratch_shapes=[
            pltpu.VMEM((1, HW), jnp.float32),
            pltpu.VMEM((1, HW), jnp.int32),
            pltpu.VMEM((1, HW), jnp.float32),
        ],
        compiler_params=pltpu.CompilerParams(
            dimension_semantics=("arbitrary", "arbitrary"),
        ),
    )(zb, cbb, fnorm, cnorm)


def _make_sc_gather(K, D, N):
    info = plsc.get_sparse_core_info()
    nw = info.num_cores * info.num_subcores
    rows_per_w = N // nw
    chunk = 128
    nchunks = rows_per_w // chunk
    mesh = plsc.VectorSubcoreMesh(core_axis_name="c", subcore_axis_name="s")

    @functools.partial(
        pl.kernel,
        out_type=jax.ShapeDtypeStruct((N, D), jnp.float32),
        mesh=mesh,
        scratch_types=[
            pltpu.VMEM((nchunks, chunk), jnp.int32),
            pltpu.VMEM((chunk, D), jnp.float32),
            pltpu.VMEM((chunk, D), jnp.float32),
            pltpu.SemaphoreType.DMA,
            pltpu.SemaphoreType.DMA,
        ],
    )
    def gather(cb_hbm, idx_hbm, out_hbm, idx_v, buf0, buf1, sem0, sem1):
        wid = lax.axis_index("s") * info.num_cores + lax.axis_index("c")
        base = wid * rows_per_w
        pltpu.sync_copy(idx_hbm.at[wid], idx_v)
        bufs = (buf0, buf1)
        sems = (sem0, sem1)
        copies = [None] * nchunks
        copies[0] = pltpu.async_copy(cb_hbm.at[idx_v.at[0]], bufs[0], sems[0])
        for j in range(nchunks):
            if j + 1 < nchunks:
                copies[j + 1] = pltpu.async_copy(
                    cb_hbm.at[idx_v.at[j + 1]], bufs[(j + 1) % 2], sems[(j + 1) % 2]
                )
            copies[j].wait()
            pltpu.sync_copy(
                bufs[j % 2], out_hbm.at[pl.ds(base + j * chunk, chunk)]
            )

    return gather


def kernel(z_e, codebook):
    B, D, H, W = z_e.shape
    HW = H * W
    N = B * HW
    K = codebook.shape[0]

    z3 = z_e.reshape(B, D, HW)
    flat = jnp.transpose(z_e, (0, 2, 3, 1)).reshape(-1, D)
    fnorm = jnp.sum(flat**2, axis=1).reshape(B, 1, HW)
    cnorm = jnp.sum(codebook**2, axis=1).reshape(K, 1)

    idx3, loss_vec = _nearest_indices(
        z3.astype(jnp.bfloat16), codebook.astype(jnp.bfloat16), fnorm, cnorm
    )

    info = plsc.get_sparse_core_info()
    nw = info.num_cores * info.num_subcores
    idx_w = idx3.reshape(nw, (N // nw) // 128, 128)
    zq_rows = _make_sc_gather(K, D, N)(codebook, idx_w)

    z_q = zq_rows.reshape(B, H, W, D).transpose(0, 3, 1, 2)
    loss = jnp.sum(loss_vec) / (N * D)
    return z_q, idx3.reshape(B, H, W), loss


# speedup vs baseline: 1.2369x; 1.0833x over previous
import functools

import jax
import jax.numpy as jnp
from jax import lax
from jax.experimental import pallas as pl
from jax.experimental.pallas import tpu as pltpu
from jax.experimental.pallas import tpu_sc as plsc

_BK = 2736
_KCB = 8192


def _bf16_rtne(x):
    u = lax.bitcast_convert_type(x, jnp.int32)
    r = (u + jnp.int32(0x7FFF) + ((u >> 16) & jnp.int32(1))) & jnp.int32(~0xFFFF)
    return lax.bitcast_convert_type(r, jnp.float32)


def _dist_argmin_kernel(
    zb_ref, cb_ref, fn_ref, cn_ref, kf_ref, idx_ref, loss_ref, minv, mina, accv
):
    b = pl.program_id(0)
    kt = pl.program_id(1)
    nk = pl.num_programs(1)
    nb = pl.num_programs(0)
    bk = cb_ref.shape[0]
    kcb = _KCB

    cross = jnp.dot(
        cb_ref[...], zb_ref[0], preferred_element_type=jnp.float32
    )
    dist = (cross * jnp.float32(-2.0) + fn_ref[0]) + cn_ref[...]
    kglob = lax.broadcasted_iota(jnp.int32, dist.shape, 0) + kt * bk
    dist = jnp.where(kglob < kcb, dist, jnp.inf)
    tmin = jnp.min(dist, axis=0, keepdims=True)
    targ_f = jnp.min(
        jnp.where(dist == tmin, kf_ref[...], jnp.float32(2.0**30)),
        axis=0,
        keepdims=True,
    )
    targ = targ_f.astype(jnp.int32)

    @pl.when(jnp.logical_and(b == 0, kt == 0))
    def _():
        accv[...] = jnp.zeros_like(accv)

    @pl.when(kt == 0)
    def _():
        minv[...] = _bf16_rtne(tmin)
        mina[...] = targ

    @pl.when(kt != 0)
    def _():
        upd = tmin < minv[...]
        merged = jnp.where(upd, tmin, minv[...])
        mina[...] = jnp.where(upd, targ, mina[...])
        minv[...] = _bf16_rtne(merged)

        @pl.when(kt == nk - 1)
        def _():
            idx_ref[0] = mina[...]
            accv[...] += merged

    @pl.when(jnp.logical_and(b == nb - 1, kt == nk - 1))
    def _():
        loss_ref[...] = accv[...]


def _nearest_indices(zb, cbb, fnorm, cnorm):
    B, D, HW = zb.shape
    K = cbb.shape[0]
    nkt = (K + _BK - 1) // _BK
    grid = (B, nkt)
    kidx_f = jnp.arange(nkt * _BK, dtype=jnp.float32).reshape(-1, 1)
    return pl.pallas_call(
        _dist_argmin_kernel,
        grid=grid,
        in_specs=[
            pl.BlockSpec((1, D, HW), lambda b, kt: (b, 0, 0)),
            pl.BlockSpec((_BK, D), lambda b, kt: (kt, 0)),
            pl.BlockSpec((1, 1, HW), lambda b, kt: (b, 0, 0)),
            pl.BlockSpec((_BK, 1), lambda b, kt: (kt, 0)),
            pl.BlockSpec((_BK, 1), lambda b, kt: (kt, 0)),
        ],
        out_specs=[
            pl.BlockSpec((1, 1, HW), lambda b, kt: (b, 0, 0)),
            pl.BlockSpec((1, HW), lambda b, kt: (0, 0)),
        ],
        out_shape=[
            jax.ShapeDtypeStruct((B, 1, HW), jnp.int32),
            jax.ShapeDtypeStruct((1, HW), jnp.float32),
        ],
        scratch_shapes=[
            pltpu.VMEM((1, HW), jnp.float32),
            pltpu.VMEM((1, HW), jnp.int32),
            pltpu.VMEM((1, HW), jnp.float32),
        ],
        compiler_params=pltpu.CompilerParams(
            dimension_semantics=("arbitrary", "arbitrary"),
        ),
    )(zb, cbb, fnorm, cnorm, kidx_f)


def _make_sc_gather(K, D, N):
    info = plsc.get_sparse_core_info()
    nw = info.num_cores * info.num_subcores
    rows_per_w = N // nw
    chunk = 128
    nchunks = rows_per_w // chunk
    mesh = plsc.VectorSubcoreMesh(core_axis_name="c", subcore_axis_name="s")

    @functools.partial(
        pl.kernel,
        out_type=jax.ShapeDtypeStruct((N, D), jnp.float32),
        mesh=mesh,
        scratch_types=[
            pltpu.VMEM((nchunks, chunk), jnp.int32),
            pltpu.VMEM((chunk, D), jnp.float32),
            pltpu.VMEM((chunk, D), jnp.float32),
            pltpu.SemaphoreType.DMA,
            pltpu.SemaphoreType.DMA,
        ],
    )
    def gather(cb_hbm, idx_hbm, out_hbm, idx_v, buf0, buf1, sem0, sem1):
        wid = lax.axis_index("s") * info.num_cores + lax.axis_index("c")
        base = wid * rows_per_w
        pltpu.sync_copy(idx_hbm.at[wid], idx_v)
        bufs = (buf0, buf1)
        sems = (sem0, sem1)
        copies = [None] * nchunks
        copies[0] = pltpu.async_copy(cb_hbm.at[idx_v.at[0]], bufs[0], sems[0])
        for j in range(nchunks):
            if j + 1 < nchunks:
                copies[j + 1] = pltpu.async_copy(
                    cb_hbm.at[idx_v.at[j + 1]], bufs[(j + 1) % 2], sems[(j + 1) % 2]
                )
            copies[j].wait()
            pltpu.sync_copy(
                bufs[j % 2], out_hbm.at[pl.ds(base + j * chunk, chunk)]
            )

    return gather


def kernel(z_e, codebook):
    B, D, H, W = z_e.shape
    HW = H * W
    N = B * HW
    K = codebook.shape[0]

    z3 = z_e.reshape(B, D, HW)
    flat = jnp.transpose(z_e, (0, 2, 3, 1)).reshape(-1, D)
    fnorm = jnp.sum(flat**2, axis=1).reshape(B, 1, HW)
    cnorm = jnp.sum(codebook**2, axis=1).reshape(K, 1)

    idx3, loss_vec = _nearest_indices(
        z3.astype(jnp.bfloat16), codebook.astype(jnp.bfloat16), fnorm, cnorm
    )

    info = plsc.get_sparse_core_info()
    nw = info.num_cores * info.num_subcores
    idx_w = idx3.reshape(nw, (N // nw) // 128, 128)
    zq_rows = _make_sc_gather(K, D, N)(codebook, idx_w)

    z_q = zq_rows.reshape(B, H, W, D).transpose(0, 3, 1, 2)
    loss = jnp.sum(loss_vec) / (N * D)
    return z_q, idx3.reshape(B, H, W), loss


# speedup vs baseline: 1.2651x; 1.0228x over previous
import functools

import jax
import jax.numpy as jnp
from jax import lax
from jax.experimental import pallas as pl
from jax.experimental.pallas import tpu as pltpu
from jax.experimental.pallas import tpu_sc as plsc

_BK = 2736
_KCB = 8192


def _bf16_rtne(x):
    u = lax.bitcast_convert_type(x, jnp.int32)
    r = (u + jnp.int32(0x7FFF) + ((u >> 16) & jnp.int32(1))) & jnp.int32(~0xFFFF)
    return lax.bitcast_convert_type(r, jnp.float32)


def _dist_argmin_kernel(
    zb_ref, cb_ref, fn_ref, cn_ref, kf_ref, idx_ref, loss_ref, minv, mina, accv
):
    b = pl.program_id(0)
    kt = pl.program_id(1)
    nk = pl.num_programs(1)
    nb = pl.num_programs(0)
    bk = cb_ref.shape[0]
    kcb = _KCB

    neg2cross = jnp.dot(
        cb_ref[...], zb_ref[0], preferred_element_type=jnp.float32
    )
    dist = (neg2cross + fn_ref[0]) + cn_ref[...]
    kglob = lax.broadcasted_iota(jnp.int32, dist.shape, 0) + kt * bk
    dist = jnp.where(kglob < kcb, dist, jnp.inf)
    tmin = jnp.min(dist, axis=0, keepdims=True)
    targ_f = jnp.min(
        jnp.where(dist == tmin, kf_ref[...], jnp.float32(2.0**30)),
        axis=0,
        keepdims=True,
    )
    targ = targ_f.astype(jnp.int32)

    @pl.when(jnp.logical_and(b == 0, kt == 0))
    def _():
        accv[...] = jnp.zeros_like(accv)

    @pl.when(kt == 0)
    def _():
        minv[...] = _bf16_rtne(tmin)
        mina[...] = targ

    @pl.when(kt != 0)
    def _():
        upd = tmin < minv[...]
        merged = jnp.where(upd, tmin, minv[...])
        mina[...] = jnp.where(upd, targ, mina[...])
        minv[...] = _bf16_rtne(merged)

        @pl.when(kt == nk - 1)
        def _():
            idx_ref[0] = mina[...]
            accv[...] += merged

    @pl.when(jnp.logical_and(b == nb - 1, kt == nk - 1))
    def _():
        loss_ref[...] = accv[...]


def _nearest_indices(zb, cbb, fnorm, cnorm):
    B, D, HW = zb.shape
    K = cbb.shape[0]
    nkt = (K + _BK - 1) // _BK
    grid = (B, nkt)
    kidx_f = jnp.arange(nkt * _BK, dtype=jnp.float32).reshape(-1, 1)
    return pl.pallas_call(
        _dist_argmin_kernel,
        grid=grid,
        in_specs=[
            pl.BlockSpec((1, D, HW), lambda b, kt: (b, 0, 0)),
            pl.BlockSpec((_BK, D), lambda b, kt: (kt, 0)),
            pl.BlockSpec((1, 1, HW), lambda b, kt: (b, 0, 0)),
            pl.BlockSpec((_BK, 1), lambda b, kt: (kt, 0)),
            pl.BlockSpec((_BK, 1), lambda b, kt: (kt, 0)),
        ],
        out_specs=[
            pl.BlockSpec((1, 1, HW), lambda b, kt: (b, 0, 0)),
            pl.BlockSpec((1, HW), lambda b, kt: (0, 0)),
        ],
        out_shape=[
            jax.ShapeDtypeStruct((B, 1, HW), jnp.int32),
            jax.ShapeDtypeStruct((1, HW), jnp.float32),
        ],
        scratch_shapes=[
            pltpu.VMEM((1, HW), jnp.float32),
            pltpu.VMEM((1, HW), jnp.int32),
            pltpu.VMEM((1, HW), jnp.float32),
        ],
        compiler_params=pltpu.CompilerParams(
            dimension_semantics=("arbitrary", "arbitrary"),
        ),
    )(zb, cbb, fnorm, cnorm, kidx_f)


def _make_sc_gather(K, D, N):
    info = plsc.get_sparse_core_info()
    nw = info.num_cores * info.num_subcores
    rows_per_w = N // nw
    chunk = 128
    nchunks = rows_per_w // chunk
    mesh = plsc.VectorSubcoreMesh(core_axis_name="c", subcore_axis_name="s")

    @functools.partial(
        pl.kernel,
        out_type=jax.ShapeDtypeStruct((N, D), jnp.float32),
        mesh=mesh,
        scratch_types=[
            pltpu.VMEM((nchunks, chunk), jnp.int32),
            pltpu.VMEM((chunk, D), jnp.float32),
            pltpu.VMEM((chunk, D), jnp.float32),
            pltpu.SemaphoreType.DMA,
            pltpu.SemaphoreType.DMA,
        ],
    )
    def gather(cb_hbm, idx_hbm, out_hbm, idx_v, buf0, buf1, sem0, sem1):
        wid = lax.axis_index("s") * info.num_cores + lax.axis_index("c")
        base = wid * rows_per_w
        pltpu.sync_copy(idx_hbm.at[wid], idx_v)
        bufs = (buf0, buf1)
        sems = (sem0, sem1)
        copies = [None] * nchunks
        copies[0] = pltpu.async_copy(cb_hbm.at[idx_v.at[0]], bufs[0], sems[0])
        for j in range(nchunks):
            if j + 1 < nchunks:
                copies[j + 1] = pltpu.async_copy(
                    cb_hbm.at[idx_v.at[j + 1]], bufs[(j + 1) % 2], sems[(j + 1) % 2]
                )
            copies[j].wait()
            pltpu.sync_copy(
                bufs[j % 2], out_hbm.at[pl.ds(base + j * chunk, chunk)]
            )

    return gather


def kernel(z_e, codebook):
    B, D, H, W = z_e.shape
    HW = H * W
    N = B * HW
    K = codebook.shape[0]

    z3 = z_e.reshape(B, D, HW)
    flat = jnp.transpose(z_e, (0, 2, 3, 1)).reshape(-1, D)
    fnorm = jnp.sum(flat**2, axis=1).reshape(B, 1, HW)
    cnorm = jnp.sum(codebook**2, axis=1).reshape(K, 1)

    cb_bf_neg2 = codebook.astype(jnp.bfloat16) * jnp.bfloat16(-2.0)
    idx3, loss_vec = _nearest_indices(
        z3.astype(jnp.bfloat16), cb_bf_neg2, fnorm, cnorm
    )

    info = plsc.get_sparse_core_info()
    nw = info.num_cores * info.num_subcores
    idx_w = idx3.reshape(nw, (N // nw) // 128, 128)
    zq_rows = _make_sc_gather(K, D, N)(codebook, idx_w)

    z_q = zq_rows.reshape(B, H, W, D).transpose(0, 3, 1, 2)
    loss = jnp.sum(loss_vec) / (N * D)
    return z_q, idx3.reshape(B, H, W), loss


# speedup vs baseline: 1.3338x; 1.0543x over previous
import functools

import jax
import jax.numpy as jnp
from jax import lax
from jax.experimental import pallas as pl
from jax.experimental.pallas import tpu as pltpu
from jax.experimental.pallas import tpu_sc as plsc

_BK = 2736
_KCB = 8192


def _bf16_rtne(x):
    u = lax.bitcast_convert_type(x, jnp.int32)
    r = (u + jnp.int32(0x7FFF) + ((u >> 16) & jnp.int32(1))) & jnp.int32(~0xFFFF)
    return lax.bitcast_convert_type(r, jnp.float32)


def _dist_argmin_kernel(
    zb_ref, cb_ref, fn_ref, cn_ref, kf_ref, idx_ref, loss_ref, minv, mina, accv
):
    b = pl.program_id(0)
    kt = pl.program_id(1)
    nk = pl.num_programs(1)
    nb = pl.num_programs(0)

    neg2cross = jnp.dot(
        cb_ref[...], zb_ref[0], preferred_element_type=jnp.float32
    )
    dist = (neg2cross + fn_ref[0]) + cn_ref[...]
    tmin = jnp.min(dist, axis=0, keepdims=True)
    targ_f = jnp.min(
        jnp.where(dist == tmin, kf_ref[...], jnp.float32(2.0**30)),
        axis=0,
        keepdims=True,
    )
    targ = targ_f.astype(jnp.int32)

    @pl.when(jnp.logical_and(b == 0, kt == 0))
    def _():
        accv[...] = jnp.zeros_like(accv)

    @pl.when(kt == 0)
    def _():
        minv[...] = _bf16_rtne(tmin)
        mina[...] = targ

    @pl.when(kt != 0)
    def _():
        upd = tmin < minv[...]
        merged = jnp.where(upd, tmin, minv[...])
        mina[...] = jnp.where(upd, targ, mina[...])
        minv[...] = _bf16_rtne(merged)

        @pl.when(kt == nk - 1)
        def _():
            idx_ref[0] = mina[...]
            accv[...] += merged

    @pl.when(jnp.logical_and(b == nb - 1, kt == nk - 1))
    def _():
        loss_ref[...] = accv[...]


def _nearest_indices(zb, cbb, fnorm, cnorm):
    B, D, HW = zb.shape
    K = cbb.shape[0]
    nkt = (K + _BK - 1) // _BK
    grid = (B, nkt)
    kidx_f = jnp.arange(nkt * _BK, dtype=jnp.float32).reshape(-1, 1)
    return pl.pallas_call(
        _dist_argmin_kernel,
        grid=grid,
        in_specs=[
            pl.BlockSpec((1, D, HW), lambda b, kt: (b, 0, 0)),
            pl.BlockSpec((_BK, D), lambda b, kt: (kt, 0)),
            pl.BlockSpec((1, 1, HW), lambda b, kt: (b, 0, 0)),
            pl.BlockSpec((_BK, 1), lambda b, kt: (kt, 0)),
            pl.BlockSpec((_BK, 1), lambda b, kt: (kt, 0)),
        ],
        out_specs=[
            pl.BlockSpec((1, 1, HW), lambda b, kt: (b, 0, 0)),
            pl.BlockSpec((1, HW), lambda b, kt: (0, 0)),
        ],
        out_shape=[
            jax.ShapeDtypeStruct((B, 1, HW), jnp.int32),
            jax.ShapeDtypeStruct((1, HW), jnp.float32),
        ],
        scratch_shapes=[
            pltpu.VMEM((1, HW), jnp.float32),
            pltpu.VMEM((1, HW), jnp.int32),
            pltpu.VMEM((1, HW), jnp.float32),
        ],
        compiler_params=pltpu.CompilerParams(
            dimension_semantics=("arbitrary", "arbitrary"),
        ),
    )(zb, cbb, fnorm, cnorm, kidx_f)


def _make_sc_gather(K, D, N):
    info = plsc.get_sparse_core_info()
    nw = info.num_cores * info.num_subcores
    rows_per_w = N // nw
    chunk = 128
    nchunks = rows_per_w // chunk
    mesh = plsc.VectorSubcoreMesh(core_axis_name="c", subcore_axis_name="s")

    @functools.partial(
        pl.kernel,
        out_type=jax.ShapeDtypeStruct((N, D), jnp.float32),
        mesh=mesh,
        scratch_types=[
            pltpu.VMEM((nchunks, chunk), jnp.int32),
            pltpu.VMEM((chunk, D), jnp.float32),
            pltpu.VMEM((chunk, D), jnp.float32),
            pltpu.SemaphoreType.DMA,
            pltpu.SemaphoreType.DMA,
        ],
    )
    def gather(cb_hbm, idx_hbm, out_hbm, idx_v, buf0, buf1, sem0, sem1):
        wid = lax.axis_index("s") * info.num_cores + lax.axis_index("c")
        base = wid * rows_per_w
        pltpu.sync_copy(idx_hbm.at[wid], idx_v)
        bufs = (buf0, buf1)
        sems = (sem0, sem1)
        copies = [None] * nchunks
        copies[0] = pltpu.async_copy(cb_hbm.at[idx_v.at[0]], bufs[0], sems[0])
        for j in range(nchunks):
            if j + 1 < nchunks:
                copies[j + 1] = pltpu.async_copy(
                    cb_hbm.at[idx_v.at[j + 1]], bufs[(j + 1) % 2], sems[(j + 1) % 2]
                )
            copies[j].wait()
            pltpu.sync_copy(
                bufs[j % 2], out_hbm.at[pl.ds(base + j * chunk, chunk)]
            )

    return gather


def kernel(z_e, codebook):
    B, D, H, W = z_e.shape
    HW = H * W
    N = B * HW
    K = codebook.shape[0]

    z3 = z_e.reshape(B, D, HW)
    flat = jnp.transpose(z_e, (0, 2, 3, 1)).reshape(-1, D)
    fnorm = jnp.sum(flat**2, axis=1).reshape(B, 1, HW)
    cnorm = jnp.sum(codebook**2, axis=1).reshape(K, 1)

    nkt = (K + _BK - 1) // _BK
    kpad = nkt * _BK - K
    cb_bf_neg2 = jnp.pad(
        codebook.astype(jnp.bfloat16) * jnp.bfloat16(-2.0), ((0, kpad), (0, 0))
    )
    cnorm_p = jnp.pad(cnorm, ((0, kpad), (0, 0)), constant_values=jnp.inf)
    idx3, loss_vec = _nearest_indices(
        z3.astype(jnp.bfloat16), cb_bf_neg2, fnorm, cnorm_p
    )

    info = plsc.get_sparse_core_info()
    nw = info.num_cores * info.num_subcores
    idx_w = idx3.reshape(nw, (N // nw) // 128, 128)
    zq_rows = _make_sc_gather(K, D, N)(codebook, idx_w)

    z_q = zq_rows.reshape(B, H, W, D).transpose(0, 3, 1, 2)
    loss = jnp.sum(loss_vec) / (N * D)
    return z_q, idx3.reshape(B, H, W), loss


# speedup vs baseline: 1.3346x; 1.0006x over previous
import functools

import jax
import jax.numpy as jnp
from jax import lax
from jax.experimental import pallas as pl
from jax.experimental.pallas import tpu as pltpu
from jax.experimental.pallas import tpu_sc as plsc

_BK = 2736
_KCB = 8192


def _bf16_rtne(x):
    u = lax.bitcast_convert_type(x, jnp.int32)
    r = (u + jnp.int32(0x7FFF) + ((u >> 16) & jnp.int32(1))) & jnp.int32(~0xFFFF)
    return lax.bitcast_convert_type(r, jnp.float32)


def _dist_argmin_kernel(
    zb_ref, cb_ref, fn_ref, cn_ref, kf_ref, idx_ref, loss_ref, minv, mina, accv
):
    b = pl.program_id(0)
    kt = pl.program_id(1)
    nk = pl.num_programs(1)
    nb = pl.num_programs(0)

    neg2cross = jnp.dot(
        cb_ref[...], zb_ref[0], preferred_element_type=jnp.float32
    )
    dist = (neg2cross + fn_ref[0]) + cn_ref[...]
    tmin = jnp.min(dist, axis=0, keepdims=True)
    targ_f = jnp.min(
        jnp.where(dist == tmin, kf_ref[...], jnp.float32(2.0**30)),
        axis=0,
        keepdims=True,
    )
    targ = targ_f.astype(jnp.int32)

    @pl.when(jnp.logical_and(b == 0, kt == 0))
    def _():
        accv[...] = jnp.zeros_like(accv)

    @pl.when(kt == 0)
    def _():
        minv[...] = _bf16_rtne(tmin)
        mina[...] = targ

    @pl.when(kt != 0)
    def _():
        upd = tmin < minv[...]
        merged = jnp.where(upd, tmin, minv[...])
        mina[...] = jnp.where(upd, targ, mina[...])
        minv[...] = _bf16_rtne(merged)

        @pl.when(kt == nk - 1)
        def _():
            idx_ref[0] = mina[...]
            accv[...] += merged

    @pl.when(jnp.logical_and(b == nb - 1, kt == nk - 1))
    def _():
        loss_ref[...] = accv[...]


def _nearest_indices(zb, cbb, fnorm, cnorm):
    B, D, HW = zb.shape
    K = cbb.shape[0]
    nkt = (K + _BK - 1) // _BK
    grid = (B, nkt)
    kidx_f = jnp.arange(nkt * _BK, dtype=jnp.float32).reshape(-1, 1)
    return pl.pallas_call(
        _dist_argmin_kernel,
        grid=grid,
        in_specs=[
            pl.BlockSpec((1, D, HW), lambda b, kt: (b, 0, 0)),
            pl.BlockSpec((_BK, D), lambda b, kt: (kt, 0)),
            pl.BlockSpec((1, 1, HW), lambda b, kt: (b, 0, 0)),
            pl.BlockSpec((_BK, 1), lambda b, kt: (kt, 0)),
            pl.BlockSpec((_BK, 1), lambda b, kt: (kt, 0)),
        ],
        out_specs=[
            pl.BlockSpec((1, 1, HW), lambda b, kt: (b, 0, 0)),
            pl.BlockSpec((1, HW), lambda b, kt: (0, 0)),
        ],
        out_shape=[
            jax.ShapeDtypeStruct((B, 1, HW), jnp.int32),
            jax.ShapeDtypeStruct((1, HW), jnp.float32),
        ],
        scratch_shapes=[
            pltpu.VMEM((1, HW), jnp.float32),
            pltpu.VMEM((1, HW), jnp.int32),
            pltpu.VMEM((1, HW), jnp.float32),
        ],
        compiler_params=pltpu.CompilerParams(
            dimension_semantics=("arbitrary", "arbitrary"),
        ),
    )(zb, cbb, fnorm, cnorm, kidx_f)


def _make_sc_gather(K, D, N):
    info = plsc.get_sparse_core_info()
    nw = info.num_cores * info.num_subcores
    rows_per_w = N // nw
    chunk = 128
    nchunks = rows_per_w // chunk
    mesh = plsc.VectorSubcoreMesh(core_axis_name="c", subcore_axis_name="s")

    @functools.partial(
        pl.kernel,
        out_type=jax.ShapeDtypeStruct((N, D), jnp.float32),
        mesh=mesh,
        scratch_types=[
            pltpu.VMEM((nchunks, chunk), jnp.int32),
            pltpu.VMEM((chunk, D), jnp.float32),
            pltpu.VMEM((chunk, D), jnp.float32),
            pltpu.SemaphoreType.DMA,
            pltpu.SemaphoreType.DMA,
            pltpu.SemaphoreType.DMA,
            pltpu.SemaphoreType.DMA,
        ],
    )
    def gather(cb_hbm, idx_hbm, out_hbm, idx_v, buf0, buf1, g0, g1, s0, s1):
        wid = lax.axis_index("s") * info.num_cores + lax.axis_index("c")
        base = wid * rows_per_w
        pltpu.sync_copy(idx_hbm.at[wid], idx_v)
        bufs = (buf0, buf1)
        gsems = (g0, g1)
        ssems = (s0, s1)
        gets = [None] * nchunks
        puts = [None] * nchunks
        gets[0] = pltpu.async_copy(cb_hbm.at[idx_v.at[0]], bufs[0], gsems[0])
        for j in range(nchunks):
            gets[j].wait()
            puts[j] = pltpu.async_copy(
                bufs[j % 2], out_hbm.at[pl.ds(base + j * chunk, chunk)],
                ssems[j % 2],
            )
            if j + 1 < nchunks:
                if j >= 1:
                    puts[j - 1].wait()
                gets[j + 1] = pltpu.async_copy(
                    cb_hbm.at[idx_v.at[j + 1]], bufs[(j + 1) % 2],
                    gsems[(j + 1) % 2],
                )
        puts[nchunks - 2].wait()
        puts[nchunks - 1].wait()

    return gather


def kernel(z_e, codebook):
    B, D, H, W = z_e.shape
    HW = H * W
    N = B * HW
    K = codebook.shape[0]

    z3 = z_e.reshape(B, D, HW)
    flat = jnp.transpose(z_e, (0, 2, 3, 1)).reshape(-1, D)
    fnorm = jnp.sum(flat**2, axis=1).reshape(B, 1, HW)
    cnorm = jnp.sum(codebook**2, axis=1).reshape(K, 1)

    nkt = (K + _BK - 1) // _BK
    kpad = nkt * _BK - K
    cb_bf_neg2 = jnp.pad(
        codebook.astype(jnp.bfloat16) * jnp.bfloat16(-2.0), ((0, kpad), (0, 0))
    )
    cnorm_p = jnp.pad(cnorm, ((0, kpad), (0, 0)), constant_values=jnp.inf)
    idx3, loss_vec = _nearest_indices(
        z3.astype(jnp.bfloat16), cb_bf_neg2, fnorm, cnorm_p
    )

    info = plsc.get_sparse_core_info()
    nw = info.num_cores * info.num_subcores
    idx_w = idx3.reshape(nw, (N // nw) // 128, 128)
    zq_rows = _make_sc_gather(K, D, N)(codebook, idx_w)

    z_q = zq_rows.reshape(B, H, W, D).transpose(0, 3, 1, 2)
    loss = jnp.sum(loss_vec) / (N * D)
    return z_q, idx3.reshape(B, H, W), loss


# speedup vs baseline: 1.3390x; 1.0033x over previous
import functools

import jax
import jax.numpy as jnp
from jax import lax
from jax.experimental import pallas as pl
from jax.experimental.pallas import tpu as pltpu
from jax.experimental.pallas import tpu_sc as plsc

_BK = 2736
_KCB = 8192


def _bf16_rtne(x):
    u = lax.bitcast_convert_type(x, jnp.int32)
    r = (u + jnp.int32(0x7FFF) + ((u >> 16) & jnp.int32(1))) & jnp.int32(~0xFFFF)
    return lax.bitcast_convert_type(r, jnp.float32)


def _dist_argmin_kernel(
    zb_ref, cb_ref, fn_ref, cn_ref, kf_ref, idx_ref, loss_ref, minv, mina, accv
):
    kt = pl.program_id(0)
    b = pl.program_id(1)
    nk = pl.num_programs(0)
    nb = pl.num_programs(1)

    neg2cross = jnp.dot(
        cb_ref[...], zb_ref[0], preferred_element_type=jnp.float32
    )
    dist = (neg2cross + fn_ref[0]) + cn_ref[...]
    tmin = jnp.min(dist, axis=0, keepdims=True)
    targ_f = jnp.min(
        jnp.where(dist == tmin, kf_ref[...], jnp.float32(2.0**30)),
        axis=0,
        keepdims=True,
    )
    targ = targ_f.astype(jnp.int32)

    @pl.when(jnp.logical_and(b == 0, kt == 0))
    def _():
        accv[...] = jnp.zeros_like(accv)

    bs = pl.ds(b, 1)

    @pl.when(kt == 0)
    def _():
        minv[bs, :] = _bf16_rtne(tmin)
        mina[bs, :] = targ

    @pl.when(kt != 0)
    def _():
        upd = tmin < minv[bs, :]
        merged = jnp.where(upd, tmin, minv[bs, :])
        mina[bs, :] = jnp.where(upd, targ, mina[bs, :])
        minv[bs, :] = _bf16_rtne(merged)

        @pl.when(kt == nk - 1)
        def _():
            idx_ref[bs, 0, :] = mina[bs, :]
            accv[...] += merged

    @pl.when(jnp.logical_and(b == nb - 1, kt == nk - 1))
    def _():
        loss_ref[...] = accv[...]


def _nearest_indices(zb, cbb, fnorm, cnorm):
    B, D, HW = zb.shape
    K = cbb.shape[0]
    nkt = (K + _BK - 1) // _BK
    grid = (nkt, B)
    kidx_f = jnp.arange(nkt * _BK, dtype=jnp.float32).reshape(-1, 1)
    return pl.pallas_call(
        _dist_argmin_kernel,
        grid=grid,
        in_specs=[
            pl.BlockSpec((1, D, HW), lambda kt, b: (b, 0, 0)),
            pl.BlockSpec((_BK, D), lambda kt, b: (kt, 0)),
            pl.BlockSpec((1, 1, HW), lambda kt, b: (b, 0, 0)),
            pl.BlockSpec((_BK, 1), lambda kt, b: (kt, 0)),
            pl.BlockSpec((_BK, 1), lambda kt, b: (kt, 0)),
        ],
        out_specs=[
            pl.BlockSpec((B, 1, HW), lambda kt, b: (0, 0, 0)),
            pl.BlockSpec((1, HW), lambda kt, b: (0, 0)),
        ],
        out_shape=[
            jax.ShapeDtypeStruct((B, 1, HW), jnp.int32),
            jax.ShapeDtypeStruct((1, HW), jnp.float32),
        ],
        scratch_shapes=[
            pltpu.VMEM((B, HW), jnp.float32),
            pltpu.VMEM((B, HW), jnp.int32),
            pltpu.VMEM((1, HW), jnp.float32),
        ],
        compiler_params=pltpu.CompilerParams(
            dimension_semantics=("arbitrary", "arbitrary"),
        ),
    )(zb, cbb, fnorm, cnorm, kidx_f)


def _make_sc_gather(K, D, N):
    info = plsc.get_sparse_core_info()
    nw = info.num_cores * info.num_subcores
    rows_per_w = N // nw
    chunk = 128
    nchunks = rows_per_w // chunk
    mesh = plsc.VectorSubcoreMesh(core_axis_name="c", subcore_axis_name="s")

    @functools.partial(
        pl.kernel,
        out_type=jax.ShapeDtypeStruct((N, D), jnp.float32),
        mesh=mesh,
        scratch_types=[
            pltpu.VMEM((nchunks, chunk), jnp.int32),
            pltpu.VMEM((chunk, D), jnp.float32),
            pltpu.VMEM((chunk, D), jnp.float32),
            pltpu.SemaphoreType.DMA,
            pltpu.SemaphoreType.DMA,
            pltpu.SemaphoreType.DMA,
            pltpu.SemaphoreType.DMA,
        ],
    )
    def gather(cb_hbm, idx_hbm, out_hbm, idx_v, buf0, buf1, g0, g1, s0, s1):
        wid = lax.axis_index("s") * info.num_cores + lax.axis_index("c")
        base = wid * rows_per_w
        pltpu.sync_copy(idx_hbm.at[wid], idx_v)
        bufs = (buf0, buf1)
        gsems = (g0, g1)
        ssems = (s0, s1)
        gets = [None] * nchunks
        puts = [None] * nchunks
        gets[0] = pltpu.async_copy(cb_hbm.at[idx_v.at[0]], bufs[0], gsems[0])
        for j in range(nchunks):
            gets[j].wait()
            puts[j] = pltpu.async_copy(
                bufs[j % 2], out_hbm.at[pl.ds(base + j * chunk, chunk)],
                ssems[j % 2],
            )
            if j + 1 < nchunks:
                if j >= 1:
                    puts[j - 1].wait()
                gets[j + 1] = pltpu.async_copy(
                    cb_hbm.at[idx_v.at[j + 1]], bufs[(j + 1) % 2],
                    gsems[(j + 1) % 2],
                )
        puts[nchunks - 2].wait()
        puts[nchunks - 1].wait()

    return gather


def kernel(z_e, codebook):
    B, D, H, W = z_e.shape
    HW = H * W
    N = B * HW
    K = codebook.shape[0]

    z3 = z_e.reshape(B, D, HW)
    flat = jnp.transpose(z_e, (0, 2, 3, 1)).reshape(-1, D)
    fnorm = jnp.sum(flat**2, axis=1).reshape(B, 1, HW)
    cnorm = jnp.sum(codebook**2, axis=1).reshape(K, 1)

    nkt = (K + _BK - 1) // _BK
    kpad = nkt * _BK - K
    cb_bf_neg2 = jnp.pad(
        codebook.astype(jnp.bfloat16) * jnp.bfloat16(-2.0), ((0, kpad), (0, 0))
    )
    cnorm_p = jnp.pad(cnorm, ((0, kpad), (0, 0)), constant_values=jnp.inf)
    idx3, loss_vec = _nearest_indices(
        z3.astype(jnp.bfloat16), cb_bf_neg2, fnorm, cnorm_p
    )

    info = plsc.get_sparse_core_info()
    nw = info.num_cores * info.num_subcores
    idx_w = idx3.reshape(nw, (N // nw) // 128, 128)
    zq_rows = _make_sc_gather(K, D, N)(codebook, idx_w)

    z_q = zq_rows.reshape(B, H, W, D).transpose(0, 3, 1, 2)
    loss = jnp.sum(loss_vec) / (N * D)
    return z_q, idx3.reshape(B, H, W), loss


# speedup vs baseline: 1.3420x; 1.0022x over previous
import functools

import jax
import jax.numpy as jnp
from jax import lax
from jax.experimental import pallas as pl
from jax.experimental.pallas import tpu as pltpu
from jax.experimental.pallas import tpu_sc as plsc

_BK = 2736


def _bf16_rtne(x):
    u = lax.bitcast_convert_type(x, jnp.int32)
    r = (u + jnp.int32(0x7FFF) + ((u >> 16) & jnp.int32(1))) & jnp.int32(~0xFFFF)
    return lax.bitcast_convert_type(r, jnp.float32)


def _dist_argmin_kernel(
    zb_ref, cb_ref, fn_ref, cn_ref, kf_ref, idx_ref, loss_ref, minv, mina, accv
):
    kt = pl.program_id(0)
    b = pl.program_id(1)
    nk = pl.num_programs(0)
    nb = pl.num_programs(1)

    neg2cross = jnp.dot(
        cb_ref[...], zb_ref[0], preferred_element_type=jnp.float32
    )
    dist = (neg2cross + fn_ref[0]) + cn_ref[...]
    tmin = jnp.min(dist, axis=0, keepdims=True)
    targ_f = jnp.min(
        jnp.where(dist == tmin, kf_ref[...], jnp.float32(2.0**30)),
        axis=0,
        keepdims=True,
    )
    targ = targ_f.astype(jnp.int32)

    @pl.when(jnp.logical_and(b == 0, kt == 0))
    def _():
        accv[...] = jnp.zeros_like(accv)

    bs = pl.ds(b, 1)

    @pl.when(kt == 0)
    def _():
        minv[bs, :] = _bf16_rtne(tmin)
        mina[bs, :] = targ

    @pl.when(kt != 0)
    def _():
        upd = tmin < minv[bs, :]
        merged = jnp.where(upd, tmin, minv[bs, :])
        mina[bs, :] = jnp.where(upd, targ, mina[bs, :])
        minv[bs, :] = _bf16_rtne(merged)

        @pl.when(kt == nk - 1)
        def _():
            idx_ref[bs, 0, :] = mina[bs, :]
            accv[...] += merged

    @pl.when(jnp.logical_and(b == nb - 1, kt == nk - 1))
    def _():
        loss_ref[...] = accv[...]


def _nearest_indices(zb, cbb, fnorm, cnorm):
    B, D, HW = zb.shape
    K = cbb.shape[0]
    nkt = (K + _BK - 1) // _BK
    grid = (nkt, B)
    kidx_f = jnp.arange(nkt * _BK, dtype=jnp.float32).reshape(-1, 1)
    return pl.pallas_call(
        _dist_argmin_kernel,
        grid=grid,
        in_specs=[
            pl.BlockSpec((1, D, HW), lambda kt, b: (b, 0, 0)),
            pl.BlockSpec((_BK, D), lambda kt, b: (kt, 0)),
            pl.BlockSpec((1, 1, HW), lambda kt, b: (b, 0, 0)),
            pl.BlockSpec((_BK, 1), lambda kt, b: (kt, 0)),
            pl.BlockSpec((_BK, 1), lambda kt, b: (kt, 0)),
        ],
        out_specs=[
            pl.BlockSpec((B, 1, HW), lambda kt, b: (0, 0, 0)),
            pl.BlockSpec((1, HW), lambda kt, b: (0, 0)),
        ],
        out_shape=[
            jax.ShapeDtypeStruct((B, 1, HW), jnp.int32),
            jax.ShapeDtypeStruct((1, HW), jnp.float32),
        ],
        scratch_shapes=[
            pltpu.VMEM((B, HW), jnp.float32),
            pltpu.VMEM((B, HW), jnp.int32),
            pltpu.VMEM((1, HW), jnp.float32),
        ],
        compiler_params=pltpu.CompilerParams(
            dimension_semantics=("arbitrary", "arbitrary"),
        ),
    )(zb, cbb, fnorm, cnorm, kidx_f)


def _make_sc_gather(K, D, N):
    info = plsc.get_sparse_core_info()
    nw = info.num_cores * info.num_subcores
    rows_per_w = N // nw
    chunk = 128
    nchunks = rows_per_w // chunk
    mesh = plsc.VectorSubcoreMesh(core_axis_name="c", subcore_axis_name="s")

    @functools.partial(
        pl.kernel,
        out_type=jax.ShapeDtypeStruct((N, D), jnp.float32),
        mesh=mesh,
        scratch_types=[
            pltpu.VMEM((nchunks, chunk), jnp.int32),
            pltpu.VMEM((chunk, D), jnp.float32),
            pltpu.VMEM((chunk, D), jnp.float32),
            pltpu.SemaphoreType.DMA,
            pltpu.SemaphoreType.DMA,
            pltpu.SemaphoreType.DMA,
            pltpu.SemaphoreType.DMA,
        ],
    )
    def gather(cb_hbm, idx_hbm, out_hbm, idx_v, buf0, buf1, g0, g1, s0, s1):
        wid = lax.axis_index("s") * info.num_cores + lax.axis_index("c")
        base = wid * rows_per_w
        pltpu.sync_copy(idx_hbm.at[wid], idx_v)
        bufs = (buf0, buf1)
        gsems = (g0, g1)
        ssems = (s0, s1)
        gets = [None] * nchunks
        puts = [None] * nchunks
        gets[0] = pltpu.async_copy(cb_hbm.at[idx_v.at[0]], bufs[0], gsems[0])
        for j in range(nchunks):
            gets[j].wait()
            puts[j] = pltpu.async_copy(
                bufs[j % 2], out_hbm.at[pl.ds(base + j * chunk, chunk)],
                ssems[j % 2],
            )
            if j + 1 < nchunks:
                if j >= 1:
                    puts[j - 1].wait()
                gets[j + 1] = pltpu.async_copy(
                    cb_hbm.at[idx_v.at[j + 1]], bufs[(j + 1) % 2],
                    gsems[(j + 1) % 2],
                )
        puts[nchunks - 2].wait()
        puts[nchunks - 1].wait()

    return gather


def kernel(z_e, codebook):
    B, D, H, W = z_e.shape
    HW = H * W
    N = B * HW
    K = codebook.shape[0]

    z3 = z_e.reshape(B, D, HW)
    flat = jnp.transpose(z_e, (0, 2, 3, 1)).reshape(-1, D)
    fnorm = jnp.sum(flat**2, axis=1).reshape(B, 1, HW)
    cnorm = jnp.sum(codebook**2, axis=1).reshape(K, 1)

    nkt = (K + _BK - 1) // _BK
    kpad = nkt * _BK - K
    cb_bf_neg2 = jnp.pad(
        codebook.astype(jnp.bfloat16) * jnp.bfloat16(-2.0), ((0, kpad), (0, 0))
    )
    cnorm_p = jnp.pad(cnorm, ((0, kpad), (0, 0)), constant_values=jnp.inf)
    idx3, loss_vec = _nearest_indices(
        z3.astype(jnp.bfloat16), cb_bf_neg2, fnorm, cnorm_p
    )

    info = plsc.get_sparse_core_info()
    nw = info.num_cores * info.num_subcores
    idx_w = idx3.reshape(nw, (N // nw) // 128, 128)
    zq_rows = _make_sc_gather(K, D, N)(codebook, idx_w)

    z_q = zq_rows.reshape(B, H, W, D).transpose(0, 3, 1, 2)
    loss = jnp.sum(loss_vec) / (N * D)
    return z_q, idx3.reshape(B, H, W), loss
